# Initial kernel scaffold; baseline (speedup 1.0000x reference)
import functools

import jax
import jax.numpy as jnp
from jax import lax
from jax.experimental import pallas as pl
from jax.experimental.pallas import tpu as pltpu
from jax.experimental.pallas import tpu_sc as plsc

N = 10000
E = 320000
D = 128
H = 128
C = 10
G = 64

NC = 2
NS = 16
NW = NC * NS
EPT = E // NW
K = 100
NCHUNK = EPT // K
ZR = 128
NPAD = 10240
ZPT = NPAD // NS
OPT = N // NS


def _sc_agg_body(x_hbm, src_hbm, dst_hbm, z_hbm, out_hbm,
                 idx_s, idx_d, rows, zbuf, agg, sem):
    c = lax.axis_index("c")
    s = lax.axis_index("s")
    tid = c * NS + s

    pltpu.sync_copy(z_hbm, zbuf)
    for z in range(ZPT // ZR):
        pltpu.sync_copy(zbuf, agg.at[pl.ds(s * ZPT + z * ZR, ZR)])

    pltpu.sync_copy(src_hbm.at[tid], idx_s)
    pltpu.sync_copy(dst_hbm.at[tid], idx_d)

    plsc.subcore_barrier()

    @pl.loop(0, NCHUNK)
    def _chunk(j):
        pltpu.async_copy(x_hbm.at[idx_s.at[j]], rows, sem).wait()
        pltpu.sync_copy(rows, agg.at[idx_d.at[j]], add=True)

    plsc.subcore_barrier()

    pltpu.sync_copy(agg.at[pl.ds(s * OPT, OPT)],
                    out_hbm.at[pl.ds(c * N + s * OPT, OPT)])


def _sc_aggregate(x, src3d, dst3d, zeros_blk):
    mesh = plsc.VectorSubcoreMesh(core_axis_name="c", subcore_axis_name="s",
                                  num_cores=NC, num_subcores=NS)
    f = pl.kernel(
        _sc_agg_body,
        out_type=jax.ShapeDtypeStruct((NC * N, D), jnp.float32),
        mesh=mesh,
        scratch_types=[
            pltpu.VMEM((NCHUNK, K), jnp.int32),
            pltpu.VMEM((NCHUNK, K), jnp.int32),
            pltpu.VMEM((K, D), jnp.float32),
            pltpu.VMEM((ZR, D), jnp.float32),
            pltpu.VMEM_SHARED((NPAD, D), jnp.float32),
            pltpu.SemaphoreType.DMA,
        ],
    )
    return f(x, src3d, dst3d, zeros_blk)


RB = 500
NRB = N // RB


def _tc_layer_body(p_ref, x_ref, wr_ref, b_ref, wo_ref, o_ref):
    agg = p_ref[0] + p_ref[1]
    acc = jax.lax.dot_general(agg, wr_ref[...], (((1,), (0,)), ((), ())),
                              preferred_element_type=jnp.float32)
    acc += jax.lax.dot_general(x_ref[...], wo_ref[...], (((1,), (0,)), ((), ())),
                               preferred_element_type=jnp.float32)
    o_ref[...] = jnp.maximum(acc + b_ref[...], 0.0)


def _tc_layer(partials, x, w_rel, b_rel, w_root):
    return pl.pallas_call(
        _tc_layer_body,
        grid=(NRB,),
        in_specs=[
            pl.BlockSpec((2, RB, D), lambda i: (0, i, 0)),
            pl.BlockSpec((RB, D), lambda i: (i, 0)),
            pl.BlockSpec((D, H), lambda i: (0, 0)),
            pl.BlockSpec((1, H), lambda i: (0, 0)),
            pl.BlockSpec((D, H), lambda i: (0, 0)),
        ],
        out_specs=pl.BlockSpec((RB, H), lambda i: (i, 0)),
        out_shape=jax.ShapeDtypeStruct((N, H), jnp.float32),
    )(partials, x, w_rel, b_rel.reshape(1, H), w_root)


def _tc_head_body(p_ref, h1_ref, wr_ref, b_ref, wo_ref, bat_ref, fw_ref,
                  fb_ref, o_ref, sums, counts):
    i = pl.program_id(0)

    @pl.when(i == 0)
    def _():
        sums[...] = jnp.zeros_like(sums)
        counts[...] = jnp.zeros_like(counts)

    agg = p_ref[0] + p_ref[1]
    acc = jax.lax.dot_general(agg, wr_ref[...], (((1,), (0,)), ((), ())),
                              preferred_element_type=jnp.float32)
    acc += jax.lax.dot_general(h1_ref[...], wo_ref[...], (((1,), (0,)), ((), ())),
                               preferred_element_type=jnp.float32)
    h2 = jnp.maximum(acc + b_ref[...], 0.0)

    bat = bat_ref[0, 0, :]
    gids = jax.lax.broadcasted_iota(jnp.int32, (G, RB), 0)
    mask = (bat[None, :] == gids).astype(jnp.float32)
    sums[...] += jax.lax.dot_general(mask, h2, (((1,), (0,)), ((), ())),
                                     preferred_element_type=jnp.float32)
    counts[...] += jax.lax.dot_general(
        mask, jnp.ones((RB, H), jnp.float32), (((1,), (0,)), ((), ())),
        preferred_element_type=jnp.float32)

    @pl.when(i == NRB - 1)
    def _():
        pooled = sums[...] / jnp.maximum(counts[...], 1.0)
        logits = jax.lax.dot_general(pooled, fw_ref[...], (((1,), (0,)), ((), ())),
                                     preferred_element_type=jnp.float32)
        logits += fb_ref[...]
        o_ref[...] = 1.0 / (1.0 + jnp.exp(-logits))


def _tc_head(partials, h1, w_rel, b_rel, w_root, batch3d, fc_w_pad, fc_b_pad):
    return pl.pallas_call(
        _tc_head_body,
        grid=(NRB,),
        in_specs=[
            pl.BlockSpec((2, RB, D), lambda i: (0, i, 0)),
            pl.BlockSpec((RB, H), lambda i: (i, 0)),
            pl.BlockSpec((H, H), lambda i: (0, 0)),
            pl.BlockSpec((1, H), lambda i: (0, 0)),
            pl.BlockSpec((H, H), lambda i: (0, 0)),
            pl.BlockSpec((1, 1, RB), lambda i: (i, 0, 0)),
            pl.BlockSpec((H, H), lambda i: (0, 0)),
            pl.BlockSpec((1, H), lambda i: (0, 0)),
        ],
        out_specs=pl.BlockSpec((G, H), lambda i: (0, 0)),
        out_shape=jax.ShapeDtypeStruct((G, H), jnp.float32),
        scratch_shapes=[
            pltpu.VMEM((G, H), jnp.float32),
            pltpu.VMEM((G, H), jnp.float32),
        ],
    )(partials, h1, w_rel, b_rel.reshape(1, H), w_root, batch3d,
      fc_w_pad, fc_b_pad)


def kernel(x, edge_index, batch, w1_rel, b1_rel, w1_root, w2_rel, b2_rel,
           w2_root, fc_w, fc_b):
    src3d = edge_index[0].reshape(NW, NCHUNK, K)
    dst3d = edge_index[1].reshape(NW, NCHUNK, K)
    zeros_blk = jnp.zeros((ZR, D), jnp.float32)
    batch3d = batch.reshape(NRB, 1, RB)
    fc_w_pad = jnp.zeros((H, H), jnp.float32).at[:, :C].set(fc_w)
    fc_b_pad = jnp.zeros((1, H), jnp.float32).at[0, :C].set(fc_b)

    p1 = _sc_aggregate(x, src3d, dst3d, zeros_blk).reshape(NC, N, D)
    h1 = _tc_layer(p1, x, w1_rel, b1_rel, w1_root)
    p2 = _sc_aggregate(h1, src3d, dst3d, zeros_blk).reshape(NC, N, D)
    out = _tc_head(p2, h1, w2_rel, b2_rel, w2_root, batch3d, fc_w_pad, fc_b_pad)
    return out[:, :C]


# speedup vs baseline: 7.8511x; 7.8511x over previous
import functools

import jax
import jax.numpy as jnp
from jax import lax
from jax.experimental import pallas as pl
from jax.experimental.pallas import tpu as pltpu
from jax.experimental.pallas import tpu_sc as plsc

N = 10000
E = 320000
D = 128
H = 128
C = 10
G = 64

NC = 2
NS = 16
NW = NC * NS
EPT = E // NW
K = 100
NCHUNK = EPT // K
ZR = 128
NPAD = 10240
ZPT = NPAD // NS
OPT = N // NS


def _sc_agg_body(x_hbm, src_hbm, dst_hbm, z_hbm, out_hbm,
                 idx_s, idx_d, rows, agg, sem):
    c = lax.axis_index("c")
    s = lax.axis_index("s")
    tid = c * NS + s

    pltpu.sync_copy(z_hbm, rows)
    for z in range(ZPT // K):
        pltpu.sync_copy(rows, agg.at[pl.ds(s * ZPT + z * K, K)])
    pltpu.sync_copy(rows.at[pl.ds(0, ZPT % K)],
                    agg.at[pl.ds(s * ZPT + (ZPT // K) * K, ZPT % K)])

    pltpu.sync_copy(src_hbm.at[tid], idx_s)
    pltpu.sync_copy(dst_hbm.at[tid], idx_d)

    plsc.subcore_barrier()

    @pl.loop(0, NCHUNK)
    def _chunk(j):
        pltpu.async_copy(x_hbm.at[idx_s.at[j]], rows, sem).wait()
        pltpu.sync_copy(rows, agg.at[idx_d.at[j]], add=True)

    plsc.subcore_barrier()

    pltpu.sync_copy(agg.at[pl.ds(s * ZPT, ZPT)],
                    out_hbm.at[pl.ds(c * NPAD + s * ZPT, ZPT)])


def _sc_aggregate(x, src3d, dst3d, zeros_blk):
    mesh = plsc.VectorSubcoreMesh(core_axis_name="c", subcore_axis_name="s",
                                  num_cores=NC, num_subcores=NS)
    f = pl.kernel(
        _sc_agg_body,
        out_type=jax.ShapeDtypeStruct((NC * NPAD, D), jnp.float32),
        mesh=mesh,
        scratch_types=[
            pltpu.VMEM((NCHUNK, K), jnp.int32),
            pltpu.VMEM((NCHUNK, K), jnp.int32),
            pltpu.VMEM((K, D), jnp.float32),
            pltpu.VMEM_SHARED((NPAD, D), jnp.float32),
            pltpu.SemaphoreType.DMA,
        ],
    )
    return f(x, src3d, dst3d, zeros_blk)


RB = 400
NRB = N // RB


def _tc_layer_body(p_ref, x_ref, wr_ref, b_ref, wo_ref, o_ref):
    agg = p_ref[0] + p_ref[1]
    acc = jax.lax.dot_general(agg, wr_ref[...], (((1,), (0,)), ((), ())),
                              preferred_element_type=jnp.float32)
    acc += jax.lax.dot_general(x_ref[...], wo_ref[...], (((1,), (0,)), ((), ())),
                               preferred_element_type=jnp.float32)
    o_ref[...] = jnp.maximum(acc + b_ref[...], 0.0)


def _tc_layer(partials, x, w_rel, b_rel, w_root):
    return pl.pallas_call(
        _tc_layer_body,
        grid=(NRB,),
        in_specs=[
            pl.BlockSpec((2, RB, D), lambda i: (0, i, 0)),
            pl.BlockSpec((RB, D), lambda i: (i, 0)),
            pl.BlockSpec((D, H), lambda i: (0, 0)),
            pl.BlockSpec((1, H), lambda i: (0, 0)),
            pl.BlockSpec((D, H), lambda i: (0, 0)),
        ],
        out_specs=pl.BlockSpec((RB, H), lambda i: (i, 0)),
        out_shape=jax.ShapeDtypeStruct((N, H), jnp.float32),
    )(partials, x, w_rel, b_rel.reshape(1, H), w_root)


def _tc_head_body(p_ref, h1_ref, wr_ref, b_ref, wo_ref, bat_ref, fw_ref,
                  fb_ref, o_ref, sums, counts):
    i = pl.program_id(0)

    @pl.when(i == 0)
    def _():
        sums[...] = jnp.zeros_like(sums)
        counts[...] = jnp.zeros_like(counts)

    agg = p_ref[0] + p_ref[1]
    acc = jax.lax.dot_general(agg, wr_ref[...], (((1,), (0,)), ((), ())),
                              preferred_element_type=jnp.float32)
    acc += jax.lax.dot_general(h1_ref[...], wo_ref[...], (((1,), (0,)), ((), ())),
                               preferred_element_type=jnp.float32)
    h2 = jnp.maximum(acc + b_ref[...], 0.0)

    bat = bat_ref[0, 0, :]
    gids = jax.lax.broadcasted_iota(jnp.int32, (G, RB), 0)
    mask = (bat[None, :] == gids).astype(jnp.float32)
    sums[...] += jax.lax.dot_general(mask, h2, (((1,), (0,)), ((), ())),
                                     preferred_element_type=jnp.float32)
    counts[...] += jax.lax.dot_general(
        mask, jnp.ones((RB, H), jnp.float32), (((1,), (0,)), ((), ())),
        preferred_element_type=jnp.float32)

    @pl.when(i == NRB - 1)
    def _():
        pooled = sums[...] / jnp.maximum(counts[...], 1.0)
        logits = jax.lax.dot_general(pooled, fw_ref[...], (((1,), (0,)), ((), ())),
                                     preferred_element_type=jnp.float32)
        logits += fb_ref[...]
        o_ref[...] = 1.0 / (1.0 + jnp.exp(-logits))


def _tc_head(partials, h1, w_rel, b_rel, w_root, batch3d, fc_w_pad, fc_b_pad):
    return pl.pallas_call(
        _tc_head_body,
        grid=(NRB,),
        in_specs=[
            pl.BlockSpec((2, RB, D), lambda i: (0, i, 0)),
            pl.BlockSpec((RB, H), lambda i: (i, 0)),
            pl.BlockSpec((H, H), lambda i: (0, 0)),
            pl.BlockSpec((1, H), lambda i: (0, 0)),
            pl.BlockSpec((H, H), lambda i: (0, 0)),
            pl.BlockSpec((1, 1, RB), lambda i: (i, 0, 0)),
            pl.BlockSpec((H, H), lambda i: (0, 0)),
            pl.BlockSpec((1, H), lambda i: (0, 0)),
        ],
        out_specs=pl.BlockSpec((G, H), lambda i: (0, 0)),
        out_shape=jax.ShapeDtypeStruct((G, H), jnp.float32),
        scratch_shapes=[
            pltpu.VMEM((G, H), jnp.float32),
            pltpu.VMEM((G, H), jnp.float32),
        ],
    )(partials, h1, w_rel, b_rel.reshape(1, H), w_root, batch3d,
      fc_w_pad, fc_b_pad)


def kernel(x, edge_index, batch, w1_rel, b1_rel, w1_root, w2_rel, b2_rel,
           w2_root, fc_w, fc_b):
    src3d = edge_index[0].reshape(NW, NCHUNK, K)
    dst3d = edge_index[1].reshape(NW, NCHUNK, K)
    zeros_blk = jnp.zeros((K, D), jnp.float32)
    batch3d = batch.reshape(NRB, 1, RB)
    fc_w_pad = jnp.zeros((H, H), jnp.float32).at[:, :C].set(fc_w)
    fc_b_pad = jnp.zeros((1, H), jnp.float32).at[0, :C].set(fc_b)

    p1 = _sc_aggregate(x, src3d, dst3d, zeros_blk).reshape(NC, NPAD, D)
    h1 = _tc_layer(p1, x, w1_rel, b1_rel, w1_root)
    p2 = _sc_aggregate(h1, src3d, dst3d, zeros_blk).reshape(NC, NPAD, D)
    out = _tc_head(p2, h1, w2_rel, b2_rel, w2_root, batch3d, fc_w_pad, fc_b_pad)
    return out[:, :C]
